# baseline (speedup 1.0000x reference)
import functools

import jax
import jax.numpy as jnp
from jax import lax
from jax.experimental import pallas as pl
from jax.experimental.pallas import tpu as pltpu
from jax.experimental.pallas import tpu_sc as plsc

_NUM_CORES = 2
_NUM_SUBCORES = 16
_NW = _NUM_CORES * _NUM_SUBCORES
_NBUF = 2
_BPC = 4


def _gather_sc(indices2, table, nb, h):
    d = table.shape[1]
    hh = h // 2
    b_per_w = nb // _NW
    n_chunks = b_per_w // _BPC
    assert n_chunks % _NBUF == 0 and n_chunks >= 2 * _NBUF
    rows_pc = _BPC * hh

    mesh = plsc.VectorSubcoreMesh(
        core_axis_name="c", subcore_axis_name="s",
        num_cores=_NUM_CORES, num_subcores=_NUM_SUBCORES,
    )

    @functools.partial(
        pl.kernel,
        mesh=mesh,
        compiler_params=pltpu.CompilerParams(use_tc_tiling_on_sc=False),
        out_type=jax.ShapeDtypeStruct((nb * hh, 2 * d), jnp.float32),
        scratch_types=[
            pltpu.VMEM((_NBUF, 2 * _BPC, hh), jnp.int32),
            pltpu.VMEM((_NBUF, 2 * _BPC, hh, d), jnp.float32),
            pltpu.SemaphoreType.DMA((_NBUF,)),
            pltpu.SemaphoreType.DMA((_NBUF,)),
        ],
    )
    def k(idx_hbm, table_hbm, out_hbm, idx_v, rows_v, gsem, osem):
        wid = lax.axis_index("s") * _NUM_CORES + lax.axis_index("c")
        base = wid * b_per_w

        def out_copies(bi, b, wait):
            for j in range(_BPC):
                for half in range(2):
                    cp = pltpu.make_async_copy(
                        rows_v.at[b, 2 * j + half],
                        out_hbm.at[pl.ds((bi + j) * hh, hh),
                                   pl.ds(half * d, d)],
                        osem.at[b],
                    )
                    if wait:
                        cp.wait()
                    else:
                        cp.start()

        def step(cc, b, wait_out):
            bi = base + cc * _BPC
            if wait_out:
                out_copies(bi, b, wait=True)
            pltpu.sync_copy(idx_hbm.at[pl.ds(2 * bi, 2 * _BPC)], idx_v.at[b])
            for jj in range(2 * _BPC):
                pltpu.async_copy(
                    table_hbm.at[idx_v.at[b, jj]],
                    rows_v.at[b, jj],
                    gsem.at[b],
                )
            for jj in range(2 * _BPC):
                pltpu.make_async_copy(
                    table_hbm.at[idx_v.at[b, jj]],
                    rows_v.at[b, jj],
                    gsem.at[b],
                ).wait()
            out_copies(bi, b, wait=False)

        for b in range(_NBUF):
            step(jnp.int32(b), b, wait_out=False)

        def body(r, carry):
            c0 = _NBUF + r * _NBUF
            for b in range(_NBUF):
                step(c0 + b, b, wait_out=True)
            return carry

        lax.fori_loop(0, n_chunks // _NBUF - 1, body, 0)

        for b in range(_NBUF):
            bi = base + (n_chunks - _NBUF + b) * _BPC
            out_copies(bi, b, wait=True)

    return k(indices2, table)


def _format_tc(flat2, nb, h, d, group):
    hh = h // 2

    def fmt_body(x_ref, o_ref):
        x = x_ref[...]
        for g in range(group):
            o_ref[g, 0:hh, :] = x[g * hh:(g + 1) * hh, 0:d]
            o_ref[g, hh:h, :] = x[g * hh:(g + 1) * hh, d:2 * d]

    return pl.pallas_call(
        fmt_body,
        grid=(nb // group,),
        in_specs=[pl.BlockSpec((group * hh, 2 * d), lambda i: (i, 0))],
        out_specs=pl.BlockSpec((group, h, d), lambda i: (i, 0, 0)),
        out_shape=jax.ShapeDtypeStruct((nb, h, d), jnp.float32),
    )(flat2)


@jax.jit
def _embed(indices, table):
    nb, h = indices.shape
    d = table.shape[1]
    idx2 = indices.reshape(2 * nb, h // 2).astype(jnp.int32)
    flat2 = _gather_sc(idx2, table, nb, h)
    return _format_tc(flat2, nb, h, d, group=16)


def kernel(indices, table):
    return _embed(indices, table)
